# baseline (speedup 1.0000x reference)
import functools

import jax
import jax.numpy as jnp
from jax import lax
from jax.experimental import pallas as pl
from jax.experimental.pallas import tpu as pltpu
from jax.experimental.pallas import tpu_sc as plsc

N = 100000
T = 64
B = 64
D = 128

NC = 2
NS = 16
NW = NC * NS
CH = 160
NCHUNK = N // CH
RG = CH // 16
MAXG = 10
HU = 4
DU = 4


def _sqrt16(x):
    i = plsc.bitcast(x, jnp.int32)
    y = plsc.bitcast(jnp.int32(0x5F3759DF) - (i >> 1), jnp.float32)
    for _ in range(3):
        y = y * (1.5 - 0.5 * x * y * y)
    return jnp.where(x > 0.0, x * y, 0.0)


def _sc_body(to_hbm, po_hbm, hid_hbm, out_hbm,
             to_b0, to_b1, po_b0, po_b1, hid_b0, hid_b1, acc, sem0, sem1):
    wid = lax.axis_index("s") * NC + lax.axis_index("c")
    niter = (NCHUNK - wid + NW - 1) // NW

    to_b = (to_b0, to_b1)
    po_b = (po_b0, po_b1)
    hid_b = (hid_b0, hid_b1)
    sems = (sem0, sem1)

    zeros16 = jnp.zeros((16,), jnp.float32)
    acc[0, :] = zeros16
    acc[1, :] = zeros16

    def issue(b, c):
        base = c * CH
        sem = sems[b]
        pltpu.async_copy(to_hbm.at[pl.ds(base, CH)], to_b[b], sem)
        pltpu.async_copy(po_hbm.at[pl.ds(base, CH)], po_b[b], sem)
        pltpu.async_copy(hid_hbm.at[pl.ds(base, CH)], hid_b[b], sem)

    def drain(b):
        sem = sems[b]
        pltpu.make_async_copy(to_hbm.at[pl.ds(0, CH)], to_b[b], sem).wait()
        pltpu.make_async_copy(po_hbm.at[pl.ds(0, CH)], po_b[b], sem).wait()
        pltpu.make_async_copy(hid_hbm.at[pl.ds(0, CH)], hid_b[b], sem).wait()

    rows_iota = lax.broadcasted_iota(jnp.int32, (16,), 0)

    def process(b):
        hidb = hid_b[b]
        tob, pob = to_b[b], po_b[b]

        def rowgroup(rg, carry):
            ao, an = carry
            r0 = rg * 16

            tov = tob[pl.ds(r0, 16)]
            pov = pob[pl.ds(r0, 16)]
            d0 = pov - tov
            ao = ao + d0 * d0

            rowv = r0 + rows_iota

            def dcol(tc, anc):
                for k in range(DU):
                    c = tc * DU + k
                    colv = (rows_iota + c) & (D - 1)
                    hv = plsc.load_gather(hidb, [rowv, colv])
                    anc = anc + hv * hv
                return anc

            ss = lax.fori_loop(0, D // DU, dcol, zeros16)
            an = an + _sqrt16(ss)
            return ao, an

        ao, an = lax.fori_loop(0, RG, rowgroup, (zeros16, zeros16))
        plsc.addupdate(acc.at[0], ao)
        plsc.addupdate(acc.at[1], an)

    issue(0, wid)
    issue(1, wid + NW)

    def gstep(g, carry):
        for bb in (0, 1):
            i = 2 * g + bb
            c = wid + i * NW

            @pl.when(i < niter)
            def _():
                drain(bb)
                process(bb)

                @pl.when(i + 2 < niter)
                def _():
                    issue(bb, c + 2 * NW)

        return carry

    lax.fori_loop(0, MAXG, gstep, 0)
    pltpu.sync_copy(acc, out_hbm.at[wid])


def _sc_losses(truth_out, pred_out, hidden):
    mesh = plsc.VectorSubcoreMesh(core_axis_name="c", subcore_axis_name="s")
    run = functools.partial(
        pl.kernel,
        out_type=jax.ShapeDtypeStruct((NW, 2, 16), jnp.float32),
        mesh=mesh,
        compiler_params=pltpu.CompilerParams(needs_layout_passes=False),
        scratch_types=[
            pltpu.VMEM((CH,), jnp.float32),
            pltpu.VMEM((CH,), jnp.float32),
            pltpu.VMEM((CH,), jnp.float32),
            pltpu.VMEM((CH,), jnp.float32),
            pltpu.VMEM((CH, D), jnp.float32),
            pltpu.VMEM((CH, D), jnp.float32),
            pltpu.VMEM((2, 16), jnp.float32),
            pltpu.SemaphoreType.DMA,
            pltpu.SemaphoreType.DMA,
        ],
    )(_sc_body)
    return run(truth_out, pred_out, hidden)


BN = 1000
G = N // BN


def _hint_body(len_ref, ba_ref, th_ref, ph_ref, out_ref):
    i = pl.program_id(0)

    @pl.when(i == 0)
    def _init():
        out_ref[...] = jnp.zeros((1, 1), jnp.float32)

    d = ph_ref[...] - th_ref[...]
    d2 = d * d
    ba = ba_ref[0]
    onehot = (lax.broadcasted_iota(jnp.int32, (B, BN), 0)
              == jnp.broadcast_to(ba, (B, BN))).astype(jnp.float32)
    p = jax.lax.dot_general(
        onehot, d2, (((1,), (0,)), ((), ())),
        preferred_element_type=jnp.float32,
        precision=jax.lax.Precision.HIGHEST)
    lenr = len_ref[...]
    tri = (lax.broadcasted_iota(jnp.int32, (T, B), 0)
           < jnp.broadcast_to(lenr, (T, B))).astype(jnp.float32)
    q = jax.lax.dot_general(
        p, tri, (((1,), (0,)), ((), ())),
        preferred_element_type=jnp.float32,
        precision=jax.lax.Precision.HIGHEST)
    eye = (lax.broadcasted_iota(jnp.int32, (B, B), 0)
           == lax.broadcasted_iota(jnp.int32, (B, B), 1)).astype(jnp.float32)
    out_ref[...] += jnp.sum(q * eye, keepdims=True)


def _hint_loss_tc(truth_hint, pred_hint, batch_assign, length):
    ba3 = batch_assign.reshape(G, 1, BN)
    len2 = length.reshape(1, B)
    (hint,) = pl.pallas_call(
        _hint_body,
        grid=(G,),
        in_specs=[
            pl.BlockSpec((1, B), lambda i: (0, 0)),
            pl.BlockSpec((1, 1, BN), lambda i: (i, 0, 0)),
            pl.BlockSpec((BN, T), lambda i: (i, 0)),
            pl.BlockSpec((BN, T), lambda i: (i, 0)),
        ],
        out_specs=[pl.BlockSpec((1, 1), lambda i: (0, 0))],
        out_shape=[jax.ShapeDtypeStruct((1, 1), jnp.float32)],
        compiler_params=pltpu.CompilerParams(
            dimension_semantics=("arbitrary",),
        ),
    )(len2, ba3, truth_hint, pred_hint)
    return hint


def kernel(truth_out, pred_out, truth_hint, pred_hint, hidden,
           edge_index, batch_assign, length):
    del edge_index
    parts = _sc_losses(truth_out, pred_out, hidden)
    hint = _hint_loss_tc(truth_hint, pred_hint, batch_assign, length)
    sums = jnp.sum(parts, axis=(0, 2))
    output_loss = (sums[0] / N).reshape(1)
    hint_loss = (hint[:, 0] / (N * T)).astype(jnp.float32)
    hidden_loss = sums[1] / N
    return (output_loss, hint_loss, hidden_loss)


# speedup vs baseline: 1.3264x; 1.3264x over previous
import functools

import jax
import jax.numpy as jnp
from jax import lax
from jax.experimental import pallas as pl
from jax.experimental.pallas import tpu as pltpu
from jax.experimental.pallas import tpu_sc as plsc

N = 100000
T = 64
B = 64
D = 128

NC = 2
NS = 16
NW = NC * NS
CH = 160
NCHUNK = N // CH
RG = CH // 16
MAXG = 10
HU = 4
DU = 4


def _sqrt16(x):
    i = plsc.bitcast(x, jnp.int32)
    y = plsc.bitcast(jnp.int32(0x5F3759DF) - (i >> 1), jnp.float32)
    for _ in range(3):
        y = y * (1.5 - 0.5 * x * y * y)
    return jnp.where(x > 0.0, x * y, 0.0)


def _sc_body(to_hbm, po_hbm, hid_hbm, out_hbm,
             to_b0, to_b1, po_b0, po_b1, hid_b0, hid_b1, acc, sem0, sem1):
    wid = lax.axis_index("s") * NC + lax.axis_index("c")
    niter = (NCHUNK - wid + NW - 1) // NW

    to_b = (to_b0, to_b1)
    po_b = (po_b0, po_b1)
    hid_b = (hid_b0, hid_b1)
    sems = (sem0, sem1)

    zeros16 = jnp.zeros((16,), jnp.float32)
    acc[0, :] = zeros16
    acc[1, :] = zeros16

    def issue(b, c):
        base = c * CH
        sem = sems[b]
        pltpu.async_copy(to_hbm.at[pl.ds(base, CH)], to_b[b], sem)
        pltpu.async_copy(po_hbm.at[pl.ds(base, CH)], po_b[b], sem)
        pltpu.async_copy(hid_hbm.at[pl.ds(base, CH)], hid_b[b], sem)

    def drain(b):
        sem = sems[b]
        pltpu.make_async_copy(to_hbm.at[pl.ds(0, CH)], to_b[b], sem).wait()
        pltpu.make_async_copy(po_hbm.at[pl.ds(0, CH)], po_b[b], sem).wait()
        pltpu.make_async_copy(hid_hbm.at[pl.ds(0, CH)], hid_b[b], sem).wait()

    rows_iota = lax.broadcasted_iota(jnp.int32, (16,), 0)

    def process(b):
        hidb = hid_b[b]
        tob, pob = to_b[b], po_b[b]

        def rowgroup(rg, carry):
            ao, an = carry
            r0 = rg * 16

            tov = tob[pl.ds(r0, 16)]
            pov = pob[pl.ds(r0, 16)]
            d0 = pov - tov
            ao = ao + d0 * d0

            rowv = r0 + rows_iota

            def dcol(tc, anc):
                for k in range(DU):
                    c = tc * DU + k
                    colv = (rows_iota + c) & (D - 1)
                    hv = plsc.load_gather(hidb, [rowv, colv])
                    anc = anc + hv * hv
                return anc

            ss = lax.fori_loop(0, D // DU, dcol, zeros16)
            an = an + _sqrt16(ss)
            return ao, an

        ao, an = lax.fori_loop(0, RG, rowgroup, (zeros16, zeros16))
        plsc.addupdate(acc.at[0], ao)
        plsc.addupdate(acc.at[1], an)

    issue(0, wid)
    issue(1, wid + NW)

    def gstep(g, carry):
        for bb in (0, 1):
            i = 2 * g + bb
            c = wid + i * NW

            @pl.when(i < niter)
            def _():
                drain(bb)
                process(bb)

                @pl.when(i + 2 < niter)
                def _():
                    issue(bb, c + 2 * NW)

        return carry

    lax.fori_loop(0, MAXG, gstep, 0)
    pltpu.sync_copy(acc, out_hbm.at[wid])


def _sc_losses(truth_out, pred_out, hidden):
    mesh = plsc.VectorSubcoreMesh(core_axis_name="c", subcore_axis_name="s")
    run = functools.partial(
        pl.kernel,
        out_type=jax.ShapeDtypeStruct((NW, 2, 16), jnp.float32),
        mesh=mesh,
        compiler_params=pltpu.CompilerParams(needs_layout_passes=False),
        scratch_types=[
            pltpu.VMEM((CH,), jnp.float32),
            pltpu.VMEM((CH,), jnp.float32),
            pltpu.VMEM((CH,), jnp.float32),
            pltpu.VMEM((CH,), jnp.float32),
            pltpu.VMEM((CH, D), jnp.float32),
            pltpu.VMEM((CH, D), jnp.float32),
            pltpu.VMEM((2, 16), jnp.float32),
            pltpu.SemaphoreType.DMA,
            pltpu.SemaphoreType.DMA,
        ],
    )(_sc_body)
    return run(truth_out, pred_out, hidden)


BN = 512
G = -(-N // BN)


def _hint_body(len_ref, ba_ref, th_ref, ph_ref, out_ref):
    i = pl.program_id(0)

    @pl.when(i == 0)
    def _init():
        out_ref[...] = jnp.zeros((1, 1), jnp.float32)

    d = ph_ref[...] - th_ref[...]
    d2 = d * d
    ba = ba_ref[...]
    onehot = (lax.broadcasted_iota(jnp.int32, (B, BN), 0)
              == jnp.broadcast_to(ba, (B, BN))).astype(jnp.float32)
    lenf = len_ref[...].astype(jnp.float32)
    thr = jax.lax.dot_general(
        lenf, onehot, (((1,), (0,)), ((), ())),
        preferred_element_type=jnp.float32)
    thri = thr.astype(jnp.int32)
    t_iota = lax.broadcasted_iota(jnp.int32, (T, BN), 0)
    gcol = i * BN + lax.broadcasted_iota(jnp.int32, (T, BN), 1)
    mask = (t_iota < jnp.broadcast_to(thri, (T, BN))) & (gcol < N)
    out_ref[...] += jnp.sum(jnp.where(mask, d2, 0.0), keepdims=True)


def _hint_loss_tc(truth_hint, pred_hint, batch_assign, length):
    tht = truth_hint.T
    pht = pred_hint.T
    ba2 = batch_assign.reshape(1, N)
    len2 = length.reshape(1, B)
    (hint,) = pl.pallas_call(
        _hint_body,
        grid=(G,),
        in_specs=[
            pl.BlockSpec((1, B), lambda i: (0, 0)),
            pl.BlockSpec((1, BN), lambda i: (0, i)),
            pl.BlockSpec((T, BN), lambda i: (0, i)),
            pl.BlockSpec((T, BN), lambda i: (0, i)),
        ],
        out_specs=[pl.BlockSpec((1, 1), lambda i: (0, 0))],
        out_shape=[jax.ShapeDtypeStruct((1, 1), jnp.float32)],
        compiler_params=pltpu.CompilerParams(
            dimension_semantics=("arbitrary",),
        ),
    )(len2, ba2, tht, pht)
    return hint


def kernel(truth_out, pred_out, truth_hint, pred_hint, hidden,
           edge_index, batch_assign, length):
    del edge_index
    parts = _sc_losses(truth_out, pred_out, hidden)
    hint = _hint_loss_tc(truth_hint, pred_hint, batch_assign, length)
    sums = jnp.sum(parts, axis=(0, 2))
    output_loss = (sums[0] / N).reshape(1)
    hint_loss = (hint[:, 0] / (N * T)).astype(jnp.float32)
    hidden_loss = sums[1] / N
    return (output_loss, hint_loss, hidden_loss)


# speedup vs baseline: 2.5485x; 1.9214x over previous
import functools

import jax
import jax.numpy as jnp
from jax import lax
from jax.experimental import pallas as pl
from jax.experimental.pallas import tpu as pltpu
from jax.experimental.pallas import tpu_sc as plsc

N = 100000
T = 64
B = 64
D = 128

NC = 2
NS = 16
NW = NC * NS
CH = 160
NCHUNK = N // CH
RG = CH // 16
MAXG = 10
HU = 4
DU = 4


def _sqrt16(x):
    i = plsc.bitcast(x, jnp.int32)
    y = plsc.bitcast(jnp.int32(0x5F3759DF) - (i >> 1), jnp.float32)
    for _ in range(3):
        y = y * (1.5 - 0.5 * x * y * y)
    return jnp.where(x > 0.0, x * y, 0.0)


def _sc_body(to_hbm, po_hbm, hid_hbm, out_hbm,
             to_b0, to_b1, po_b0, po_b1, hid_b0, hid_b1, acc, sem0, sem1):
    wid = lax.axis_index("s") * NC + lax.axis_index("c")
    niter = (NCHUNK - wid + NW - 1) // NW

    to_b = (to_b0, to_b1)
    po_b = (po_b0, po_b1)
    hid_b = (hid_b0, hid_b1)
    sems = (sem0, sem1)

    zeros16 = jnp.zeros((16,), jnp.float32)
    acc[0, :] = zeros16
    acc[1, :] = zeros16

    def issue(b, c):
        base = c * CH
        sem = sems[b]
        pltpu.async_copy(to_hbm.at[pl.ds(base, CH)], to_b[b], sem)
        pltpu.async_copy(po_hbm.at[pl.ds(base, CH)], po_b[b], sem)
        pltpu.async_copy(hid_hbm.at[pl.ds(base, CH)], hid_b[b], sem)

    def drain(b):
        sem = sems[b]
        pltpu.make_async_copy(to_hbm.at[pl.ds(0, CH)], to_b[b], sem).wait()
        pltpu.make_async_copy(po_hbm.at[pl.ds(0, CH)], po_b[b], sem).wait()
        pltpu.make_async_copy(hid_hbm.at[pl.ds(0, CH)], hid_b[b], sem).wait()

    rows_iota = lax.broadcasted_iota(jnp.int32, (16,), 0)

    def process(b):
        hidb = hid_b[b]
        tob, pob = to_b[b], po_b[b]

        def rowgroup(rg, carry):
            ao, an = carry
            r0 = rg * 16

            tov = tob[pl.ds(r0, 16)]
            pov = pob[pl.ds(r0, 16)]
            d0 = pov - tov
            ao = ao + d0 * d0

            rowv = r0 + rows_iota

            def dcol(tc, anc):
                for k in range(DU):
                    c = tc * DU + k
                    colv = (rows_iota + c) & (D - 1)
                    hv = plsc.load_gather(hidb, [rowv, colv])
                    anc = anc + hv * hv
                return anc

            ss = lax.fori_loop(0, D // DU, dcol, zeros16)
            an = an + _sqrt16(ss)
            return ao, an

        ao, an = lax.fori_loop(0, RG, rowgroup, (zeros16, zeros16))
        plsc.addupdate(acc.at[0], ao)
        plsc.addupdate(acc.at[1], an)

    issue(0, wid)
    issue(1, wid + NW)

    def gstep(g, carry):
        for bb in (0, 1):
            i = 2 * g + bb
            c = wid + i * NW

            @pl.when(i < niter)
            def _():
                drain(bb)
                process(bb)

                @pl.when(i + 2 < niter)
                def _():
                    issue(bb, c + 2 * NW)

        return carry

    lax.fori_loop(0, MAXG, gstep, 0)
    pltpu.sync_copy(acc, out_hbm.at[wid])


def _sc_losses(truth_out, pred_out, hidden):
    mesh = plsc.VectorSubcoreMesh(core_axis_name="c", subcore_axis_name="s")
    run = functools.partial(
        pl.kernel,
        out_type=jax.ShapeDtypeStruct((NW, 2, 16), jnp.float32),
        mesh=mesh,
        compiler_params=pltpu.CompilerParams(needs_layout_passes=False),
        scratch_types=[
            pltpu.VMEM((CH,), jnp.float32),
            pltpu.VMEM((CH,), jnp.float32),
            pltpu.VMEM((CH,), jnp.float32),
            pltpu.VMEM((CH,), jnp.float32),
            pltpu.VMEM((CH, D), jnp.float32),
            pltpu.VMEM((CH, D), jnp.float32),
            pltpu.VMEM((2, 16), jnp.float32),
            pltpu.SemaphoreType.DMA,
            pltpu.SemaphoreType.DMA,
        ],
    )(_sc_body)
    return run(truth_out, pred_out, hidden)


BN = 2048
G = -(-N // BN)


def _hint_body(len_ref, ba_ref, th_ref, ph_ref, out_ref):
    i = pl.program_id(0)

    @pl.when(i == 0)
    def _init():
        out_ref[...] = jnp.zeros((1, 1), jnp.float32)

    d = ph_ref[...] - th_ref[...]
    gcol = i * BN + lax.broadcasted_iota(jnp.int32, (T, BN), 1)
    d2 = jnp.where(gcol < N, d * d, 0.0)
    ba = ba_ref[...]
    onehot = (lax.broadcasted_iota(jnp.int32, (B, BN), 0)
              == jnp.broadcast_to(ba, (B, BN))).astype(jnp.float32)
    p = jax.lax.dot_general(
        d2, onehot, (((1,), (1,)), ((), ())),
        preferred_element_type=jnp.float32,
        precision=jax.lax.Precision.HIGHEST)
    lenr = len_ref[...]
    tri = (lax.broadcasted_iota(jnp.int32, (T, B), 0)
           < jnp.broadcast_to(lenr, (T, B))).astype(jnp.float32)
    out_ref[...] += jnp.sum(p * tri, keepdims=True)


def _hint_loss_tc(truth_hint, pred_hint, batch_assign, length):
    tht = truth_hint.T
    pht = pred_hint.T
    ba2 = batch_assign.reshape(1, N)
    len2 = length.reshape(1, B)
    (hint,) = pl.pallas_call(
        _hint_body,
        grid=(G,),
        in_specs=[
            pl.BlockSpec((1, B), lambda i: (0, 0)),
            pl.BlockSpec((1, BN), lambda i: (0, i)),
            pl.BlockSpec((T, BN), lambda i: (0, i)),
            pl.BlockSpec((T, BN), lambda i: (0, i)),
        ],
        out_specs=[pl.BlockSpec((1, 1), lambda i: (0, 0))],
        out_shape=[jax.ShapeDtypeStruct((1, 1), jnp.float32)],
        compiler_params=pltpu.CompilerParams(
            dimension_semantics=("arbitrary",),
        ),
    )(len2, ba2, tht, pht)
    return hint


def kernel(truth_out, pred_out, truth_hint, pred_hint, hidden,
           edge_index, batch_assign, length):
    del edge_index
    parts = _sc_losses(truth_out, pred_out, hidden)
    hint = _hint_loss_tc(truth_hint, pred_hint, batch_assign, length)
    sums = jnp.sum(parts, axis=(0, 2))
    output_loss = (sums[0] / N).reshape(1)
    hint_loss = (hint[:, 0] / (N * T)).astype(jnp.float32)
    hidden_loss = sums[1] / N
    return (output_loss, hint_loss, hidden_loss)
